# baseline (speedup 1.0000x reference)
import functools

import jax
import jax.numpy as jnp
from jax import lax
from jax.experimental import pallas as pl
from jax.experimental.pallas import tpu as pltpu
from jax.experimental.pallas import tpu_sc as plsc

_N = 10000
_E = 320000
_G = 64
_NC = 2
_NCU = 1
_NS = 16
_NW = _NCU * _NS
_CH = 128
_K = 160
_EP = _NW * _K * _CH
_NPAD = 10112
_RPT = _NPAD // _NS

_CP = pltpu.CompilerParams(use_tc_tiling_on_sc=False)


def _mesh():
    return plsc.VectorSubcoreMesh(core_axis_name="c", subcore_axis_name="s",
                                  num_cores=_NCU)


@functools.cache
def _deg_kernel():
    _W = 4

    def body(dst_hbm, ones_hbm, zeros_hbm, out_hbm, dstv, onesv, acc, sem):
        c = lax.axis_index("c")
        s = lax.axis_index("s")
        w = c * _NS + s
        pltpu.sync_copy(dst_hbm.at[w], dstv)
        pltpu.sync_copy(ones_hbm, onesv)
        pltpu.sync_copy(zeros_hbm.at[pl.ds(s * _RPT, _RPT)],
                        acc.at[pl.ds(s * _RPT, _RPT)])
        plsc.subcore_barrier()

        def step(j, carry):
            @pl.when(j >= _W)
            def _():
                pltpu.make_async_copy(onesv, acc.at[dstv.at[j]], sem).wait()

            pltpu.async_copy(onesv, acc.at[dstv.at[j]], sem, add=True)
            return carry

        lax.fori_loop(0, _K, step, 0)

        def dstep(j, carry):
            pltpu.make_async_copy(onesv, acc.at[dstv.at[j]], sem).wait()
            return carry

        lax.fori_loop(0, _W, dstep, 0)
        plsc.subcore_barrier()
        pltpu.sync_copy(acc.at[pl.ds(s * _RPT, _RPT)],
                        out_hbm.at[c, pl.ds(s * _RPT, _RPT)])

    return pl.kernel(
        body,
        out_type=jax.ShapeDtypeStruct((_NCU, _NPAD, 8), jnp.float32),
        mesh=_mesh(),
        scratch_types=[
            pltpu.VMEM((_K, _CH), jnp.int32),
            pltpu.VMEM((_CH, 8), jnp.float32),
            pltpu.VMEM_SHARED((_NPAD, 8), jnp.float32),
            pltpu.SemaphoreType.DMA,
        ],
        compiler_params=_CP,
    )


@functools.cache
def _prop_kernel(d):
    def body(table_hbm, src_hbm, dst_hbm, zeros_hbm, out_hbm,
             srcv, dstv, rows_a, rows_b, acc, gsa, gsb, ssa, ssb):
        c = lax.axis_index("c")
        s = lax.axis_index("s")
        w = c * _NS + s
        pltpu.sync_copy(src_hbm.at[w], srcv)
        pltpu.sync_copy(dst_hbm.at[w], dstv)
        pltpu.sync_copy(zeros_hbm.at[pl.ds(s * _RPT, _RPT)],
                        acc.at[pl.ds(s * _RPT, _RPT)])
        plsc.subcore_barrier()

        pltpu.async_copy(table_hbm.at[srcv.at[0]], rows_a, gsa)

        def step(i, carry):
            j0 = 2 * i
            j1 = j0 + 1
            pltpu.make_async_copy(table_hbm.at[srcv.at[j0]], rows_a, gsa).wait()
            pltpu.async_copy(rows_a, acc.at[dstv.at[j0]], ssa, add=True)

            @pl.when(i > 0)
            def _():
                pltpu.make_async_copy(rows_b, acc.at[dstv.at[j1 - 2]],
                                      ssb).wait()

            pltpu.async_copy(table_hbm.at[srcv.at[j1]], rows_b, gsb)
            pltpu.make_async_copy(table_hbm.at[srcv.at[j1]], rows_b, gsb).wait()
            pltpu.async_copy(rows_b, acc.at[dstv.at[j1]], ssb, add=True)
            pltpu.make_async_copy(rows_a, acc.at[dstv.at[j0]], ssa).wait()

            @pl.when(j1 + 1 < _K)
            def _():
                pltpu.async_copy(table_hbm.at[srcv.at[j1 + 1]], rows_a, gsa)

            return carry

        lax.fori_loop(0, _K // 2, step, 0)
        pltpu.make_async_copy(rows_b, acc.at[dstv.at[_K - 1]], ssb).wait()
        plsc.subcore_barrier()
        pltpu.sync_copy(acc.at[pl.ds(s * _RPT, _RPT)],
                        out_hbm.at[c, pl.ds(s * _RPT, _RPT)])

    return pl.kernel(
        body,
        out_type=jax.ShapeDtypeStruct((_NCU, _NPAD, d), jnp.float32),
        mesh=_mesh(),
        scratch_types=[
            pltpu.VMEM((_K, _CH), jnp.int32),
            pltpu.VMEM((_K, _CH), jnp.int32),
            pltpu.VMEM((_CH, d), jnp.float32),
            pltpu.VMEM((_CH, d), jnp.float32),
            pltpu.VMEM_SHARED((_NPAD, d), jnp.float32),
            pltpu.SemaphoreType.DMA,
            pltpu.SemaphoreType.DMA,
            pltpu.SemaphoreType.DMA,
            pltpu.SemaphoreType.DMA,
        ],
        compiler_params=_CP,
    )


@functools.cache
def _prep_call():
    def body(degp, xpad, t1, deg):
        dsum = degp[0][:, 0:1] + 1.0
        deg[...] = dsum
        t1[...] = xpad[...] * lax.rsqrt(dsum)

    return pl.pallas_call(
        body,
        out_shape=(jax.ShapeDtypeStruct((_NPAD, 8), jnp.float32),
                   jax.ShapeDtypeStruct((_NPAD, 1), jnp.float32)),
    )


@functools.cache
def _layer_call(d_in, d_out, relu):
    def body(parts, hprev, deg, W, b, hout, tout):
        dsum = deg[...]
        dis = lax.rsqrt(dsum)
        ph = dis * parts[0] + hprev[...] / dsum
        h = jnp.dot(ph, W[...], preferred_element_type=jnp.float32) + b[...]
        if relu:
            h = jnp.maximum(h, 0.0)
        hout[...] = h
        tout[...] = dis * h

    return pl.pallas_call(
        body,
        out_shape=(jax.ShapeDtypeStruct((_NPAD, d_out), jnp.float32),
                   jax.ShapeDtypeStruct((_NPAD, d_out), jnp.float32)),
    )


@functools.cache
def _final_call():
    def body(parts, hprev, deg, W3, b3, Wl, bl, bat, out, pooled):
        dsum = deg[...]
        dis = lax.rsqrt(dsum)
        ph = dis * parts[0] + hprev[...] / dsum
        h3 = jnp.dot(ph, W3[...], preferred_element_type=jnp.float32) + b3[...]
        bvec = bat[...]
        neg = jnp.float32(-jnp.inf)

        def gstep(g, carry):
            col = jnp.max(jnp.where(bvec == g, h3, neg), axis=0)
            pooled[pl.ds(g, 1), :] = col[None, :]
            return carry

        lax.fori_loop(0, _G, gstep, 0)
        logits = jnp.dot(pooled[...], Wl[...],
                         preferred_element_type=jnp.float32) + bl[...]
        mx = jnp.max(logits, axis=1, keepdims=True)
        lse = mx + jnp.log(jnp.sum(jnp.exp(logits - mx), axis=1, keepdims=True))
        out[...] = logits - lse

    return pl.pallas_call(
        body,
        out_shape=jax.ShapeDtypeStruct((_G, 3), jnp.float32),
        scratch_shapes=[pltpu.VMEM((_G, 128), jnp.float32)],
    )


def kernel(x, edge_index, batch, W1, b1, W2, b2, W3, b3, Wl, bl):
    f32 = jnp.float32
    src = edge_index[0]
    dst = edge_index[1]
    fill = jnp.full((_EP - _E,), _N, jnp.int32)
    src3 = jnp.concatenate([src, fill]).reshape(_NW, _K, _CH)
    dst3 = jnp.concatenate([dst, fill]).reshape(_NW, _K, _CH)
    xpad = jnp.zeros((_NPAD, 8), f32).at[:_N, :2].set(x)
    W1p = jnp.zeros((8, 8), f32).at[:2].set(W1)
    batp = jnp.concatenate(
        [batch, jnp.full((_NPAD - _N,), _G, jnp.int32)]).reshape(_NPAD, 1)

    degp = _deg_kernel()(dst3, jnp.ones((_CH, 8), f32),
                         jnp.zeros((_NPAD, 8), f32))
    t1, deg = _prep_call()(degp, xpad)
    p1 = _prop_kernel(8)(t1, src3, dst3, jnp.zeros((_NPAD, 8), f32))
    h1, t2 = _layer_call(8, 8, True)(p1, xpad, deg, W1p, b1.reshape(1, 8))
    p2 = _prop_kernel(8)(t2, src3, dst3, jnp.zeros((_NPAD, 8), f32))
    h2, t3 = _layer_call(8, 32, True)(p2, h1, deg, W2, b2.reshape(1, 32))
    p3 = _prop_kernel(32)(t3, src3, dst3, jnp.zeros((_NPAD, 32), f32))
    return _final_call()(p3, h2, deg, W3, b3.reshape(1, 128), Wl,
                         bl.reshape(1, 3), batp)


# speedup vs baseline: 1.6402x; 1.6402x over previous
import functools

import jax
import jax.numpy as jnp
from jax import lax
from jax.experimental import pallas as pl
from jax.experimental.pallas import tpu as pltpu
from jax.experimental.pallas import tpu_sc as plsc

_N = 10000
_E = 320000
_G = 64
_NC = 2
_NS = 16
_NW = _NC * _NS
_CH = 128
_K = 80
_EP = _NW * _K * _CH
_NPAD = 10112
_RPT = _NPAD // _NS

_CP = pltpu.CompilerParams(use_tc_tiling_on_sc=False)


def _mesh():
    return plsc.VectorSubcoreMesh(core_axis_name="c", subcore_axis_name="s")


@functools.cache
def _deg_kernel():
    _W = 4

    def body(dst_hbm, ones_hbm, zeros_hbm, out_hbm, dstv, onesv, acc, sem):
        c = lax.axis_index("c")
        s = lax.axis_index("s")
        w = c * _NS + s
        pltpu.sync_copy(dst_hbm.at[w], dstv)
        pltpu.sync_copy(ones_hbm, onesv)
        pltpu.sync_copy(zeros_hbm.at[pl.ds(s * _RPT, _RPT)],
                        acc.at[pl.ds(s * _RPT, _RPT)])
        plsc.subcore_barrier()

        def step(j, carry):
            @pl.when(j >= _W)
            def _():
                pltpu.make_async_copy(onesv, acc.at[dstv.at[j]], sem).wait()

            pltpu.async_copy(onesv, acc.at[dstv.at[j]], sem, add=True)
            return carry

        lax.fori_loop(0, _K, step, 0)

        def dstep(j, carry):
            pltpu.make_async_copy(onesv, acc.at[dstv.at[j]], sem).wait()
            return carry

        lax.fori_loop(0, _W, dstep, 0)
        plsc.subcore_barrier()
        pltpu.sync_copy(acc.at[pl.ds(s * _RPT, _RPT)],
                        out_hbm.at[c, pl.ds(s * _RPT, _RPT)])

    return pl.kernel(
        body,
        out_type=jax.ShapeDtypeStruct((_NC, _NPAD, 8), jnp.float32),
        mesh=_mesh(),
        scratch_types=[
            pltpu.VMEM((_K, _CH), jnp.int32),
            pltpu.VMEM((_CH, 8), jnp.float32),
            pltpu.VMEM_SHARED((_NPAD, 8), jnp.float32),
            pltpu.SemaphoreType.DMA,
        ],
        compiler_params=_CP,
    )


@functools.cache
def _prop_kernel(d):
    _NB = 8
    _D = 4

    def body(table_hbm, src_hbm, dst_hbm, zeros_hbm, out_hbm,
             srcv, dstv, *rest):
        rows = rest[:_NB]
        acc = rest[_NB]
        gs = rest[_NB + 1:2 * _NB + 1]
        ss = rest[2 * _NB + 1:]
        c = lax.axis_index("c")
        s = lax.axis_index("s")
        w = c * _NS + s
        pltpu.sync_copy(src_hbm.at[w], srcv)
        pltpu.sync_copy(dst_hbm.at[w], dstv)
        pltpu.sync_copy(zeros_hbm.at[pl.ds(s * _RPT, _RPT)],
                        acc.at[pl.ds(s * _RPT, _RPT)])
        plsc.subcore_barrier()

        for b in range(_D):
            pltpu.async_copy(table_hbm.at[srcv.at[b]], rows[b], gs[b])

        def step(i, carry):
            for b in range(_NB):
                j = _NB * i + b
                pltpu.make_async_copy(table_hbm.at[srcv.at[j]], rows[b],
                                      gs[b]).wait()
                pltpu.async_copy(rows[b], acc.at[dstv.at[j]], ss[b], add=True)
                jf = j + _D
                bf = (b + _D) % _NB

                @pl.when(jf < _K)
                def _():
                    @pl.when(jf >= _NB)
                    def _():
                        pltpu.make_async_copy(rows[bf], acc.at[dstv.at[j]],
                                              ss[bf]).wait()

                    pltpu.async_copy(table_hbm.at[srcv.at[jf]], rows[bf],
                                     gs[bf])

            return carry

        lax.fori_loop(0, _K // _NB, step, 0)
        for b in range(_NB):
            pltpu.make_async_copy(rows[b], acc.at[dstv.at[0]], ss[b]).wait()
        plsc.subcore_barrier()
        pltpu.sync_copy(acc.at[pl.ds(s * _RPT, _RPT)],
                        out_hbm.at[c, pl.ds(s * _RPT, _RPT)])

    return pl.kernel(
        body,
        out_type=jax.ShapeDtypeStruct((_NC, _NPAD, d), jnp.float32),
        mesh=_mesh(),
        scratch_types=(
            [pltpu.VMEM((_K, _CH), jnp.int32),
             pltpu.VMEM((_K, _CH), jnp.int32)]
            + [pltpu.VMEM((_CH, d), jnp.float32) for _ in range(_NB)]
            + [pltpu.VMEM_SHARED((_NPAD, d), jnp.float32)]
            + [pltpu.SemaphoreType.DMA for _ in range(2 * _NB)]
        ),
        compiler_params=_CP,
    )


@functools.cache
def _prep_call():
    def body(degp, xpad, t1, deg):
        dsum = degp[0][:, 0:1] + degp[1][:, 0:1] + 1.0
        deg[...] = dsum
        t1[...] = xpad[...] * lax.rsqrt(dsum)

    return pl.pallas_call(
        body,
        out_shape=(jax.ShapeDtypeStruct((_NPAD, 8), jnp.float32),
                   jax.ShapeDtypeStruct((_NPAD, 1), jnp.float32)),
    )


@functools.cache
def _layer_call(d_in, d_out, relu):
    def body(parts, hprev, deg, W, b, hout, tout):
        dsum = deg[...]
        dis = lax.rsqrt(dsum)
        ph = dis * (parts[0] + parts[1]) + hprev[...] / dsum
        h = jnp.dot(ph, W[...], preferred_element_type=jnp.float32) + b[...]
        if relu:
            h = jnp.maximum(h, 0.0)
        hout[...] = h
        tout[...] = dis * h

    return pl.pallas_call(
        body,
        out_shape=(jax.ShapeDtypeStruct((_NPAD, d_out), jnp.float32),
                   jax.ShapeDtypeStruct((_NPAD, d_out), jnp.float32)),
    )


@functools.cache
def _final_call():
    def body(parts, hprev, deg, W3, b3, Wl, bl, bat, out, pooled):
        dsum = deg[...]
        dis = lax.rsqrt(dsum)
        ph = dis * (parts[0] + parts[1]) + hprev[...] / dsum
        h3 = jnp.dot(ph, W3[...], preferred_element_type=jnp.float32) + b3[...]
        bvec = bat[...]
        neg = jnp.float32(-jnp.inf)

        def gstep(g, carry):
            col = jnp.max(jnp.where(bvec == g, h3, neg), axis=0)
            pooled[pl.ds(g, 1), :] = col[None, :]
            return carry

        lax.fori_loop(0, _G, gstep, 0)
        logits = jnp.dot(pooled[...], Wl[...],
                         preferred_element_type=jnp.float32) + bl[...]
        mx = jnp.max(logits, axis=1, keepdims=True)
        lse = mx + jnp.log(jnp.sum(jnp.exp(logits - mx), axis=1, keepdims=True))
        out[...] = logits - lse

    return pl.pallas_call(
        body,
        out_shape=jax.ShapeDtypeStruct((_G, 3), jnp.float32),
        scratch_shapes=[pltpu.VMEM((_G, 128), jnp.float32)],
    )


def kernel(x, edge_index, batch, W1, b1, W2, b2, W3, b3, Wl, bl):
    f32 = jnp.float32
    src = edge_index[0]
    dst = edge_index[1]
    fill = jnp.full((_EP - _E,), _N, jnp.int32)
    src3 = jnp.concatenate([src, fill]).reshape(_NW, _K, _CH)
    dst3 = jnp.concatenate([dst, fill]).reshape(_NW, _K, _CH)
    xpad = jnp.zeros((_NPAD, 8), f32).at[:_N, :2].set(x)
    W1p = jnp.zeros((8, 8), f32).at[:2].set(W1)
    batp = jnp.concatenate(
        [batch, jnp.full((_NPAD - _N,), _G, jnp.int32)]).reshape(_NPAD, 1)

    degp = _deg_kernel()(dst3, jnp.ones((_CH, 8), f32),
                         jnp.zeros((_NPAD, 8), f32))
    t1, deg = _prep_call()(degp, xpad)
    p1 = _prop_kernel(8)(t1, src3, dst3, jnp.zeros((_NPAD, 8), f32))
    h1, t2 = _layer_call(8, 8, True)(p1, xpad, deg, W1p, b1.reshape(1, 8))
    p2 = _prop_kernel(8)(t2, src3, dst3, jnp.zeros((_NPAD, 8), f32))
    h2, t3 = _layer_call(8, 32, True)(p2, h1, deg, W2, b2.reshape(1, 32))
    p3 = _prop_kernel(32)(t3, src3, dst3, jnp.zeros((_NPAD, 32), f32))
    return _final_call()(p3, h2, deg, W3, b3.reshape(1, 128), Wl,
                         bl.reshape(1, 3), batp)
